# Initial kernel scaffold; baseline (speedup 1.0000x reference)
import jax
import jax.numpy as jnp
from jax.experimental import pallas as pl


def kernel(h, edge_attr, edge_index, W1, b1, W2, b2):
    raise NotImplementedError("write your pallas kernel here")


# speedup vs baseline: 2.0301x; 2.0301x over previous
import functools

import jax
import jax.numpy as jnp
from jax import lax
from jax.experimental import pallas as pl
from jax.experimental.pallas import tpu as pltpu
from jax.experimental.pallas import tpu_sc as plsc

LN2 = 0.6931471805599453


def _sc_gather(table, idx, chunk):
    V, D = table.shape
    B = idx.shape[0]
    mesh = plsc.VectorSubcoreMesh(core_axis_name="c", subcore_axis_name="s")
    nw = 32
    b_per_w = B // nw
    n_iter = b_per_w // chunk

    @functools.partial(
        pl.kernel,
        mesh=mesh,
        out_type=jax.ShapeDtypeStruct((B, D), jnp.float32),
        scratch_types=[
            pltpu.VMEM((chunk,), jnp.int32),
            pltpu.VMEM((chunk, D), jnp.float32),
            pltpu.SemaphoreType.DMA,
        ],
    )
    def gather_kernel(table_hbm, idx_hbm, out_hbm, idx_v, rows_v, sem):
        wid = lax.axis_index("s") * 2 + lax.axis_index("c")
        base = wid * b_per_w

        @pl.loop(0, n_iter)
        def _(it):
            off = base + it * chunk
            pltpu.sync_copy(idx_hbm.at[pl.ds(off, chunk)], idx_v)
            pltpu.async_copy(table_hbm.at[idx_v], rows_v, sem).wait()
            pltpu.sync_copy(rows_v, out_hbm.at[pl.ds(off, chunk)])

    return gather_kernel(table, idx)


def _mlp_body(h1_ref, h2_ref, ea_ref, w1a_ref, w1b_ref, w1c_ref, b1_ref,
              w2_ref, b2_ref, o_ref):
    x = jnp.dot(h1_ref[...], w1a_ref[...], preferred_element_type=jnp.float32)
    x += jnp.dot(h2_ref[...], w1b_ref[...], preferred_element_type=jnp.float32)
    x += jnp.dot(ea_ref[...], w1c_ref[...], preferred_element_type=jnp.float32)
    x += b1_ref[...]
    x = jnp.maximum(x, 0.0) + jnp.log1p(jnp.exp(-jnp.abs(x))) - LN2
    o_ref[...] = (
        jnp.dot(x, w2_ref[...], preferred_element_type=jnp.float32)
        + b2_ref[...]
    )


def _tc_mlp(hh, edge_attr, W1, b1, W2, b2, block):
    E = edge_attr.shape[0]
    d_feat = hh.shape[1]
    d_edge = edge_attr.shape[1]
    two_c = W1.shape[1]
    C = W2.shape[1]
    n_blocks = E // block

    w1a = W1[:d_feat]
    w1b = W1[d_feat:2 * d_feat]
    w1c = W1[2 * d_feat:]
    b1r = b1.reshape(1, two_c)
    b2r = b2.reshape(1, C)

    return pl.pallas_call(
        _mlp_body,
        grid=(n_blocks,),
        in_specs=[
            pl.BlockSpec((block, d_feat), lambda i: (i, 0)),
            pl.BlockSpec((block, d_feat), lambda i: (i + n_blocks, 0)),
            pl.BlockSpec((block, d_edge), lambda i: (i, 0)),
            pl.BlockSpec((d_feat, two_c), lambda i: (0, 0)),
            pl.BlockSpec((d_feat, two_c), lambda i: (0, 0)),
            pl.BlockSpec((d_edge, two_c), lambda i: (0, 0)),
            pl.BlockSpec((1, two_c), lambda i: (0, 0)),
            pl.BlockSpec((two_c, C), lambda i: (0, 0)),
            pl.BlockSpec((1, C), lambda i: (0, 0)),
        ],
        out_specs=pl.BlockSpec((block, C), lambda i: (i, 0)),
        out_shape=jax.ShapeDtypeStruct((E, C), jnp.float32),
    )(hh, hh, edge_attr, w1a, w1b, w1c, b1r, W2, b2r)


def kernel(h, edge_attr, edge_index, W1, b1, W2, b2):
    E = edge_attr.shape[0]
    idx = edge_index.astype(jnp.int32).reshape(2 * E)
    hh = _sc_gather(h, idx, chunk=400)
    return _tc_mlp(hh, edge_attr, W1, b1, W2, b2, block=512)
